# baseline (speedup 1.0000x reference)
import functools

import jax
import jax.numpy as jnp
from jax import lax
from jax.experimental import pallas as pl
from jax.experimental.pallas import tpu as pltpu
from jax.experimental.pallas import tpu_sc as plsc

_NC = 2
_NS = 16
_NW = _NC * _NS
_CH = 112
_NPAD = 10112


def _sc_aggregate(h, pk_r):
    N, D = h.shape
    NCH = pk_r.shape[1] - 1
    CH = pk_r.shape[2]
    RPT = _NPAD // _NS
    ZR = 8

    mesh = plsc.VectorSubcoreMesh(core_axis_name="c", subcore_axis_name="s")

    @functools.partial(
        pl.kernel,
        out_type=jax.ShapeDtypeStruct((_NC, N, D), jnp.float32),
        mesh=mesh,
        scratch_types=[
            pltpu.VMEM_SHARED((_NPAD, D), jnp.float32),
            pltpu.VMEM((NCH + 1, CH), jnp.int32),
            pltpu.VMEM((1, CH), jnp.int32),
            pltpu.VMEM((1, CH), jnp.int32),
            pltpu.VMEM((1, CH), jnp.int32),
            pltpu.VMEM((1, CH), jnp.int32),
            pltpu.VMEM((CH, D), jnp.float32),
            pltpu.VMEM((CH, D), jnp.float32),
            pltpu.SemaphoreType.DMA,
            pltpu.SemaphoreType.DMA,
        ],
    )
    def agg_kernel(h_hbm, pk_hbm, out_hbm,
                   acc, pk_v, ss0, ss1, sd0, sd1, rows0, rows1, sg0, sg1):
        c = lax.axis_index("c")
        s = lax.axis_index("s")
        wid = c * _NS + s

        zero = jnp.zeros((16,), jnp.float32)
        for i in range(ZR):
            for j in range(D // 16):
                rows0[i, pl.ds(j * 16, 16)] = zero
        for k in range(RPT // ZR):
            pltpu.sync_copy(rows0.at[pl.ds(0, ZR)],
                            acc.at[pl.ds(s * RPT + k * ZR, ZR)])
        plsc.subcore_barrier()

        pltpu.sync_copy(pk_hbm.at[wid], pk_v)

        ssrc = (ss0, ss1)
        sdst = (sd0, sd1)
        rbufs = (rows0, rows1)
        gsems = (sg0, sg1)

        def unpack(i, q):
            for j in range(CH // 16):
                v = pk_v[i, pl.ds(j * 16, 16)]
                ssrc[q][0, pl.ds(j * 16, 16)] = v & 0xFFFF
                sdst[q][0, pl.ds(j * 16, 16)] = v >> 16

        def gather_start(q):
            pltpu.async_copy(h_hbm.at[ssrc[q].at[0]], rbufs[q], gsems[q])

        def gather_wait(p):
            pltpu.make_async_copy(h_hbm.at[ssrc[p].at[0]], rbufs[p],
                                  gsems[p]).wait()

        def scatter(p):
            pltpu.sync_copy(rbufs[p], acc.at[sdst[p].at[0]], add=True)

        unpack(0, 0)
        gather_start(0)

        def step(i, p, q):
            unpack(i + 1, q)
            gather_start(q)
            gather_wait(p)
            scatter(p)

        def pipe(k, carry):
            step(2 * k, 0, 1)
            step(2 * k + 1, 1, 0)
            return carry

        lax.fori_loop(0, NCH // 2, pipe, 0)
        gather_wait(0)
        plsc.subcore_barrier()

        last_full = N - (_NS - 1) * RPT

        @pl.when(s < _NS - 1)
        def _():
            pltpu.sync_copy(acc.at[pl.ds(s * RPT, RPT)],
                            out_hbm.at[c, pl.ds(s * RPT, RPT)])

        @pl.when(s == _NS - 1)
        def _():
            pltpu.sync_copy(acc.at[pl.ds((_NS - 1) * RPT, last_full)],
                            out_hbm.at[c, pl.ds((_NS - 1) * RPT, last_full)])

    return agg_kernel(h, pk_r)


def _dense(h, parts, W_msg, Wu_h, Wu_a, b):
    N, D = h.shape
    BLK = 400
    dn = (((1,), (1,)), ((), ()))

    def body(h_ref, p_ref, wm_ref, wh_ref, wa_ref, b_ref, o_ref):
        g = p_ref[0] + p_ref[1]
        agg = lax.dot_general(g, wm_ref[...], dn,
                              preferred_element_type=jnp.float32)
        acc = lax.dot_general(h_ref[...], wh_ref[...], dn,
                              preferred_element_type=jnp.float32)
        acc = acc + lax.dot_general(agg, wa_ref[...], dn,
                                    preferred_element_type=jnp.float32)
        o_ref[...] = jnp.maximum(acc + b_ref[...], 0.0)

    return pl.pallas_call(
        body,
        grid=(N // BLK,),
        in_specs=[
            pl.BlockSpec((BLK, D), lambda i: (i, 0)),
            pl.BlockSpec((_NC, BLK, D), lambda i: (0, i, 0)),
            pl.BlockSpec((D, D), lambda i: (0, 0)),
            pl.BlockSpec((D, D), lambda i: (0, 0)),
            pl.BlockSpec((D, D), lambda i: (0, 0)),
            pl.BlockSpec((1, D), lambda i: (0, 0)),
        ],
        out_specs=pl.BlockSpec((BLK, D), lambda i: (i, 0)),
        out_shape=jax.ShapeDtypeStruct((N, D), jnp.float32),
    )(h, parts, W_msg, Wu_h, Wu_a, b)


def kernel(h, edge_index, W_msg, W_upd, b_upd):
    N, D = h.shape
    E = edge_index.shape[1]
    src = edge_index[0].astype(jnp.int32)
    dst = edge_index[1].astype(jnp.int32)

    epw = -(-E // (_NW * 2 * _CH)) * 2 * _CH
    e_pad = _NW * epw
    n_extra = e_pad - E
    if n_extra:
        src = jnp.concatenate([src, jnp.zeros((n_extra,), jnp.int32)])
        dst = jnp.concatenate(
            [dst, N + (jnp.arange(n_extra, dtype=jnp.int32) % (_NPAD - N))])
    nch = epw // _CH
    pk_r = ((dst << 16) | src).reshape(_NW, nch, _CH)
    pk_r = jnp.pad(pk_r, ((0, 0), (0, 1), (0, 0)))

    parts = _sc_aggregate(h, pk_r)
    return _dense(h, parts, W_msg, W_upd[:, :D], W_upd[:, D:],
                  b_upd.reshape(1, D))


# speedup vs baseline: 1.6864x; 1.6864x over previous
import functools

import jax
import jax.numpy as jnp
from jax import lax
from jax.experimental import pallas as pl
from jax.experimental.pallas import tpu as pltpu
from jax.experimental.pallas import tpu_sc as plsc

_NC = 2
_NS = 16
_NW = _NC * _NS
_CH = 125
_NPAD = 10240


def _sc_aggregate(h, src_r, dst_r):
    N, D = h.shape
    _, NCH, CH = src_r.shape
    RPT = _NPAD // _NS
    ZR = 32

    mesh = plsc.VectorSubcoreMesh(core_axis_name="c", subcore_axis_name="s")

    @functools.partial(
        pl.kernel,
        out_type=jax.ShapeDtypeStruct((_NC, N, D), jnp.float32),
        mesh=mesh,
        scratch_types=[
            pltpu.VMEM_SHARED((_NPAD, D), jnp.float32),
            pltpu.VMEM((NCH, CH), jnp.int32),
            pltpu.VMEM((NCH, CH), jnp.int32),
            pltpu.VMEM((CH, D), jnp.float32),
            pltpu.VMEM((ZR, D), jnp.float32),
            pltpu.SemaphoreType.DMA,
        ],
    )
    def agg_kernel(h_hbm, src_hbm, dst_hbm, out_hbm,
                   acc, src_v, dst_v, rows_v, zbuf, sem):
        c = lax.axis_index("c")
        s = lax.axis_index("s")
        wid = c * _NS + s

        zero = jnp.zeros((16,), jnp.float32)
        for i in range(ZR):
            for j in range(D // 16):
                zbuf[i, pl.ds(j * 16, 16)] = zero
        for k in range(RPT // ZR):
            pltpu.sync_copy(zbuf, acc.at[pl.ds(s * RPT + k * ZR, ZR)])
        plsc.subcore_barrier()

        pltpu.sync_copy(src_hbm.at[wid], src_v)
        pltpu.sync_copy(dst_hbm.at[wid], dst_v)

        def chunk(i, carry):
            pltpu.async_copy(h_hbm.at[src_v.at[i]], rows_v, sem).wait()
            pltpu.sync_copy(rows_v, acc.at[dst_v.at[i]], add=True)
            return carry

        lax.fori_loop(0, NCH, chunk, 0)
        plsc.subcore_barrier()

        last_full = N - (_NS - 1) * RPT

        @pl.when(s < _NS - 1)
        def _():
            pltpu.sync_copy(acc.at[pl.ds(s * RPT, RPT)],
                            out_hbm.at[c, pl.ds(s * RPT, RPT)])

        @pl.when(s == _NS - 1)
        def _():
            pltpu.sync_copy(acc.at[pl.ds((_NS - 1) * RPT, last_full)],
                            out_hbm.at[c, pl.ds((_NS - 1) * RPT, last_full)])

    return agg_kernel(h, src_r, dst_r)


def _dense(h, parts, W_msg, Wu_h, Wu_a, b):
    N, D = h.shape
    BLK = 400
    dn = (((1,), (1,)), ((), ()))

    def body(h_ref, p_ref, wm_ref, wh_ref, wa_ref, b_ref, o_ref):
        g = p_ref[0] + p_ref[1]
        agg = lax.dot_general(g, wm_ref[...], dn,
                              preferred_element_type=jnp.float32)
        acc = lax.dot_general(h_ref[...], wh_ref[...], dn,
                              preferred_element_type=jnp.float32)
        acc = acc + lax.dot_general(agg, wa_ref[...], dn,
                                    preferred_element_type=jnp.float32)
        o_ref[...] = jnp.maximum(acc + b_ref[...], 0.0)

    return pl.pallas_call(
        body,
        grid=(N // BLK,),
        in_specs=[
            pl.BlockSpec((BLK, D), lambda i: (i, 0)),
            pl.BlockSpec((_NC, BLK, D), lambda i: (0, i, 0)),
            pl.BlockSpec((D, D), lambda i: (0, 0)),
            pl.BlockSpec((D, D), lambda i: (0, 0)),
            pl.BlockSpec((D, D), lambda i: (0, 0)),
            pl.BlockSpec((1, D), lambda i: (0, 0)),
        ],
        out_specs=pl.BlockSpec((BLK, D), lambda i: (i, 0)),
        out_shape=jax.ShapeDtypeStruct((N, D), jnp.float32),
    )(h, parts, W_msg, Wu_h, Wu_a, b)


def kernel(h, edge_index, W_msg, W_upd, b_upd):
    N, D = h.shape
    E = edge_index.shape[1]
    src = edge_index[0].astype(jnp.int32)
    dst = edge_index[1].astype(jnp.int32)

    epw = -(-E // (_NW * _CH)) * _CH
    e_pad = _NW * epw
    n_extra = e_pad - E
    if n_extra:
        src = jnp.concatenate([src, jnp.zeros((n_extra,), jnp.int32)])
        dst = jnp.concatenate(
            [dst, N + (jnp.arange(n_extra, dtype=jnp.int32) % (_NPAD - N))])
    nch = epw // _CH
    src_r = src.reshape(_NW, nch, _CH)
    dst_r = dst.reshape(_NW, nch, _CH)

    parts = _sc_aggregate(h, src_r, dst_r)
    return _dense(h, parts, W_msg, W_upd[:, :D], W_upd[:, D:],
                  b_upd.reshape(1, D))
